# baseline (speedup 1.0000x reference)
import jax
import jax.numpy as jnp
from jax.experimental import pallas as pl
from jax.experimental.pallas import tpu as pltpu

_CH = 512
_NB = 8


def _body(idx_ref, arr_any, elem_ref, out_any, bufs, insems, outsems):
    M = arr_any.shape[0]
    nch = M // _CH
    idx = idx_ref[0]
    owner = idx // _CH
    local = idx - owner * _CH

    def in_copy(k):
        b = k % _NB
        return pltpu.make_async_copy(
            arr_any.at[pl.ds(k * _CH, _CH)], bufs.at[b], insems.at[b])

    def out_copy(k):
        b = k % _NB
        return pltpu.make_async_copy(
            bufs.at[b], out_any.at[pl.ds(k * _CH, _CH)], outsems.at[b])

    for k in range(_NB):
        in_copy(k).start()
    for k in range(nch):
        b = k % _NB
        in_copy(k).wait()

        @pl.when(owner == k)
        def _(b=b):
            pltpu.make_async_copy(
                elem_ref, bufs.at[b, pl.ds(local, 1)], insems.at[b]).start()
            pltpu.make_async_copy(
                elem_ref, bufs.at[b, pl.ds(local, 1)], insems.at[b]).wait()

        out_copy(k).start()
        if k + _NB < nch:
            out_copy(k).wait()
            in_copy(k + _NB).start()
    for k in range(nch - _NB, nch):
        if k >= 0:
            out_copy(k).wait()


def kernel(arr, index, element):
    M, N = arr.shape
    idx = jnp.asarray(index, jnp.int32).reshape((1,))
    elem2d = element.reshape((1, N))
    return pl.pallas_call(
        _body,
        grid_spec=pltpu.PrefetchScalarGridSpec(
            num_scalar_prefetch=1,
            grid=(1,),
            in_specs=[
                pl.BlockSpec(memory_space=pl.ANY),
                pl.BlockSpec((1, N), lambda i, idx_ref: (0, 0)),
            ],
            out_specs=pl.BlockSpec(memory_space=pl.ANY),
            scratch_shapes=[
                pltpu.VMEM((_NB, _CH, N), jnp.float32),
                pltpu.SemaphoreType.DMA((_NB,)),
                pltpu.SemaphoreType.DMA((_NB,)),
            ],
        ),
        out_shape=jax.ShapeDtypeStruct((M, N), arr.dtype),
    )(idx, arr, elem2d)


# speedup vs baseline: 1.0053x; 1.0053x over previous
import jax
import jax.numpy as jnp
from jax.experimental import pallas as pl
from jax.experimental.pallas import tpu as pltpu

_CH = 1024
_NB = 4


def _body(idx_ref, arr_any, elem_ref, out_any, bufs, insems, outsems):
    M = arr_any.shape[0]
    nch = M // _CH
    idx = idx_ref[0]
    owner = idx // _CH
    local = idx - owner * _CH

    def in_copy(k):
        b = k % _NB
        return pltpu.make_async_copy(
            arr_any.at[pl.ds(k * _CH, _CH)], bufs.at[b], insems.at[b])

    def out_copy(k):
        b = k % _NB
        return pltpu.make_async_copy(
            bufs.at[b], out_any.at[pl.ds(k * _CH, _CH)], outsems.at[b])

    for k in range(_NB):
        in_copy(k).start()
    for k in range(nch):
        b = k % _NB
        in_copy(k).wait()

        @pl.when(owner == k)
        def _(b=b):
            pltpu.make_async_copy(
                elem_ref, bufs.at[b, pl.ds(local, 1)], insems.at[b]).start()
            pltpu.make_async_copy(
                elem_ref, bufs.at[b, pl.ds(local, 1)], insems.at[b]).wait()

        out_copy(k).start()
        if k + _NB < nch:
            out_copy(k).wait()
            in_copy(k + _NB).start()
    for k in range(nch - _NB, nch):
        if k >= 0:
            out_copy(k).wait()


def kernel(arr, index, element):
    M, N = arr.shape
    idx = jnp.asarray(index, jnp.int32).reshape((1,))
    elem2d = element.reshape((1, N))
    return pl.pallas_call(
        _body,
        grid_spec=pltpu.PrefetchScalarGridSpec(
            num_scalar_prefetch=1,
            grid=(1,),
            in_specs=[
                pl.BlockSpec(memory_space=pl.ANY),
                pl.BlockSpec((1, N), lambda i, idx_ref: (0, 0)),
            ],
            out_specs=pl.BlockSpec(memory_space=pl.ANY),
            scratch_shapes=[
                pltpu.VMEM((_NB, _CH, N), jnp.float32),
                pltpu.SemaphoreType.DMA((_NB,)),
                pltpu.SemaphoreType.DMA((_NB,)),
            ],
        ),
        out_shape=jax.ShapeDtypeStruct((M, N), arr.dtype),
    )(idx, arr, elem2d)
